# baseline (speedup 1.0000x reference)
import functools

import jax
import jax.numpy as jnp
from jax import lax
from jax.experimental import pallas as pl
from jax.experimental.pallas import tpu as pltpu
from jax.experimental.pallas import tpu_sc as plsc

_GB = 4
_HPAD = 24
_CC = 256


def _sc_info():
    info = plsc.get_sparse_core_info()
    return info, info.num_cores * info.num_subcores, info.num_lanes


@functools.lru_cache(maxsize=None)
def _make_repack(V, D):
    info, nw, lanes = _sc_info()
    D2 = 2 * D
    full = (V // _CC) * _CC
    n_chunks = full // _CC
    tail = V - full
    per_w = n_chunks // nw
    extra = n_chunks - per_w * nw
    mesh = plsc.VectorSubcoreMesh(core_axis_name="c", subcore_axis_name="s")

    @functools.partial(
        pl.kernel,
        mesh=mesh,
        out_type=jax.ShapeDtypeStruct((V // 2, D2), jnp.float32),
        compiler_params=pltpu.CompilerParams(
            use_tc_tiling_on_sc=True, needs_layout_passes=False),
        scratch_types=[
            pltpu.VMEM((D, _CC), jnp.float32),
            pltpu.VMEM((D, _CC), jnp.float32),
            pltpu.VMEM((_CC // 2, D2), jnp.float32),
            pltpu.SemaphoreType.DMA,
            pltpu.SemaphoreType.DMA,
            pltpu.SemaphoreType.DMA,
        ],
    )
    def repack_kernel(wt_hbm, tail_hbm, p_hbm, slab0, slab1, obuf,
                      isem0, isem1, osem):
        wid = lax.axis_index("s") * info.num_cores + lax.axis_index("c")
        lane = lax.iota(jnp.int32, lanes)
        jrows = [(k * lanes) % D + lane for k in range(D2 // lanes)]

        def valid(t):
            return t * nw + wid < n_chunks

        def in_copy(t, slab, sem):
            c = t * nw + wid
            off = pl.multiple_of(c * _CC, _CC)
            return pltpu.make_async_copy(
                wt_hbm.at[:, pl.ds(off, _CC)], slab, sem)

        def out_copy(t):
            c = t * nw + wid
            off = pl.multiple_of(c * (_CC // 2), _CC // 2)
            return pltpu.make_async_copy(
                obuf, p_hbm.at[pl.ds(off, _CC // 2)], osem)

        def transpose(slab, np_rows):
            def pb(p, carry):
                col0 = 2 * p
                for k in range(D2 // lanes):
                    col = col0 + (1 if k * lanes >= D else 0)
                    v = plsc.load_gather(
                        slab, [jrows[k], jnp.full((lanes,), col, jnp.int32)])
                    obuf[p, pl.ds(k * lanes, lanes)] = v
                return carry

            lax.fori_loop(0, np_rows, pb, 0)

        if tail:
            @pl.when(wid == 0)
            def _():
                pltpu.sync_copy(tail_hbm, obuf.at[pl.ds(0, tail // 2)])
                pltpu.sync_copy(obuf.at[pl.ds(0, tail // 2)],
                                p_hbm.at[pl.ds(full // 2, tail // 2)])

        def step(t, slab, isem):
            @pl.when(valid(t))
            def _():
                in_copy(t, slab, isem).wait()

            @pl.when(jnp.logical_and(t >= 1, valid(t - 1)))
            def _():
                out_copy(t - 1).wait()

            @pl.when(valid(t))
            def _():
                transpose(slab, _CC // 2)

            @pl.when(valid(t + 2))
            def _():
                in_copy(t + 2, slab, isem).start()

            @pl.when(valid(t))
            def _():
                out_copy(t).start()

        @pl.when(valid(0))
        def _():
            in_copy(0, slab0, isem0).start()

        @pl.when(valid(1))
        def _():
            in_copy(1, slab1, isem1).start()

        def body(u, carry):
            t = 2 * u
            step(t, slab0, isem0)
            step(t + 1, slab1, isem1)
            return carry

        lax.fori_loop(0, (per_w + 1 + 2 + 1) // 2 + 1, body, 0)

    return repack_kernel


@functools.lru_cache(maxsize=None)
def _make_gather(VP, B, H, D):
    info, nw, lanes = _sc_info()
    assert D % lanes == 0 and B % nw == 0 and H <= _HPAD
    b_per_w = B // nw
    assert b_per_w % (_GB * 2) == 0 and b_per_w % lanes == 0
    n_groups = b_per_w // _GB
    rows_g = _GB * H
    assert H % 2 == 0 and rows_g % 16 == 0
    mesh = plsc.VectorSubcoreMesh(core_axis_name="c", subcore_axis_name="s")

    @functools.partial(
        pl.kernel,
        mesh=mesh,
        out_type=jax.ShapeDtypeStruct((B * H // 2, 2 * D), jnp.float32),
        compiler_params=pltpu.CompilerParams(
            use_tc_tiling_on_sc=True, needs_layout_passes=False),
        scratch_types=[
            pltpu.VMEM((H, b_per_w), jnp.int32),
            pltpu.VMEM((b_per_w, _HPAD), jnp.int32),
            pltpu.VMEM((b_per_w * 8 + lanes,), jnp.int32),
            pltpu.VMEM((_GB, _HPAD, 2 * D), jnp.float32),
            pltpu.VMEM((_GB, _HPAD, 2 * D), jnp.float32),
            pltpu.VMEM((rows_g // 2, 2 * D), jnp.float32),
            pltpu.VMEM((rows_g // 2, 2 * D), jnp.float32),
            pltpu.SemaphoreType.DMA,
            pltpu.SemaphoreType.DMA,
            pltpu.SemaphoreType.DMA,
            pltpu.SemaphoreType.DMA,
        ],
    )
    def gather_kernel(ids_hbm, table_hbm, out_hbm, idx_hw, idx_bw, par_bw,
                      pbuf0, pbuf1, stage0, stage1, gsem0, gsem1, ssem0, ssem1):
        wid = lax.axis_index("s") * info.num_cores + lax.axis_index("c")
        base_b = wid * b_per_w
        prow_base = base_b * H // 2
        pltpu.sync_copy(
            ids_hbm.at[:, pl.ds(pl.multiple_of(base_b, b_per_w), b_per_w)],
            idx_hw)

        lane = lax.iota(jnp.int32, lanes)
        zeros = jnp.zeros((lanes,), jnp.int32)

        def zf(r, carry):
            idx_bw[r, pl.ds(0, lanes)] = zeros
            idx_bw[r, pl.ds(_HPAD - lanes, lanes)] = zeros
            return carry

        lax.fori_loop(0, b_per_w, zf, 0)

        def zf2(r, carry):
            par_bw[pl.ds(r * lanes, lanes)] = zeros
            return carry

        lax.fori_loop(0, (b_per_w * 8 + lanes) // lanes, zf2, 0)

        def tr_h(h, carry):
            def tr_c(c, carry2):
                v = idx_hw[h, pl.ds(c * lanes, lanes)]
                rows = c * lanes + lane
                hvec = jnp.full((lanes,), h, jnp.int32)
                plsc.store_scatter(idx_bw, [rows, hvec], v >> 1)
                plsc.addupdate_scatter(par_bw, [rows * 8], (v & 1) << h)
                return carry2

            return lax.fori_loop(0, b_per_w // lanes, tr_c, carry)

        lax.fori_loop(0, H, tr_h, 0)

        def g_copy(g, bb, buf, sem):
            l = g * _GB + bb
            return pltpu.make_async_copy(
                table_hbm.at[idx_bw.at[l]], buf.at[bb], sem)

        def fire(g, buf, sem):
            def fb(bb, carry):
                g_copy(g, bb, buf, sem).start()
                return carry

            lax.fori_loop(0, _GB, fb, 0)

        def wait_g(g, buf, sem):
            def wb(bb, carry):
                g_copy(g, bb, buf, sem).wait()
                return carry

            lax.fori_loop(0, _GB, wb, 0)

        def select(g, buf, stage):
            def sb(bb, carry):
                l = g * _GB + bb
                pw = par_bw[pl.ds(l * 8, lanes)][0]
                for h in range(H):
                    par = ((pw >> h) & 1) * D
                    r = bb * H + h
                    cb = (h & 1) * D
                    for k in range(D // lanes):
                        stage[r >> 1, pl.ds(cb + k * lanes, lanes)] = (
                            buf[bb, h, pl.ds(par + k * lanes, lanes)])
                return carry

            lax.fori_loop(0, _GB, sb, 0)

        def s_copy(g, stage, sem):
            off = pl.multiple_of(prow_base + g * (rows_g // 2), rows_g // 2)
            return pltpu.make_async_copy(
                stage, out_hbm.at[pl.ds(off, rows_g // 2)], sem)

        def step(g, buf, stage, gsem, ssem, first, last):
            wait_g(g, buf, gsem)
            if not first:
                s_copy(g - 2, stage, ssem).wait()
            select(g, buf, stage)
            if not last:
                fire(g + 2, buf, gsem)
            s_copy(g, stage, ssem).start()

        fire(0, pbuf0, gsem0)
        fire(1, pbuf1, gsem1)
        step(0, pbuf0, stage0, gsem0, ssem0, True, False)
        step(1, pbuf1, stage1, gsem1, ssem1, True, False)

        def body(t, carry):
            a = 2 * t
            step(a, pbuf0, stage0, gsem0, ssem0, False, False)
            step(a + 1, pbuf1, stage1, gsem1, ssem1, False, False)
            return carry

        lax.fori_loop(1, n_groups // 2 - 1, body, 0)

        a = n_groups - 2
        step(a, pbuf0, stage0, gsem0, ssem0, False, True)
        step(a + 1, pbuf1, stage1, gsem1, ssem1, False, True)
        s_copy(a, stage0, ssem0).wait()
        s_copy(a + 1, stage1, ssem1).wait()

    return gather_kernel


def kernel(ids, weight):
    bsz, hist = ids.shape
    vocab, dim = weight.shape
    wt = weight.T
    full = (vocab // _CC) * _CC
    tail_packed = weight[full:].reshape((vocab - full) // 2, 2 * dim)
    pair_table = _make_repack(vocab, dim)(wt, tail_packed)
    ids_t = ids.T
    out = _make_gather(vocab // 2, bsz, hist, dim)(ids_t, pair_table)
    return out.reshape(bsz, hist, dim)
